# baseline (speedup 1.0000x reference)
import functools

import jax
import jax.numpy as jnp
from jax import lax
from jax.experimental import pallas as pl
from jax.experimental.pallas import tpu as pltpu
from jax.experimental.pallas import tpu_sc as plsc

VOCAB = 1_000_000
D = 32
B = 4096
L = 200

NC = 2
NS = 16
NW = NC * NS
BW = B // NW

_mesh = plsc.VectorSubcoreMesh(core_axis_name="c", subcore_axis_name="s")


@functools.partial(
    pl.kernel,
    out_type=[
        jax.ShapeDtypeStruct((L, D, B), jnp.float32),
        jax.ShapeDtypeStruct((D, B), jnp.float32),
    ],
    mesh=_mesh,
    compiler_params=pltpu.CompilerParams(
        use_tc_tiling_on_sc=False, needs_layout_passes=False
    ),
    scratch_types=[
        pltpu.VMEM((L, BW), jnp.int32),
        pltpu.VMEM((BW, D), jnp.float32),
        pltpu.VMEM((BW, D), jnp.float32),
        pltpu.VMEM((D, BW), jnp.float32),
        pltpu.VMEM((D, BW), jnp.float32),
        pltpu.VMEM((D, BW), jnp.float32),
        pltpu.SemaphoreType.DMA,
        pltpu.SemaphoreType.DMA,
        pltpu.SemaphoreType.DMA,
        pltpu.SemaphoreType.DMA,
    ],
)
def _sc_embed(ids_hbm, table_hbm, seq_hbm, pool_hbm,
              idsv, rows0, rows1, col0, col1, accv, sg0, sg1, sw0, sw1):
    wid = lax.axis_index("s") * NC + lax.axis_index("c")
    b0 = wid * BW

    pltpu.sync_copy(ids_hbm.at[:, pl.ds(b0, BW)], idsv)

    zero = jnp.zeros((16,), jnp.float32)

    def zero_body(i, _):
        accv[i // (BW // 16), pl.ds((i % (BW // 16)) * 16, 16)] = zero
        return 0

    lax.fori_loop(0, D * BW // 16, zero_body, 0)

    base16 = lax.iota(jnp.int32, 16)
    rowidx = [base16 + (g * 16) for g in range(BW // 16)]

    def fire_gather(t, rows_b, sem):
        pltpu.async_copy(table_hbm.at[idsv.at[t]], rows_b, sem)

    def drain_gather(rows_b, sem):
        pltpu.make_async_copy(table_hbm.at[idsv.at[0]], rows_b, sem).wait()

    def fire_out(t, col_b, sem):
        pltpu.async_copy(col_b, seq_hbm.at[t, :, pl.ds(b0, BW)], sem)

    def drain_out(col_b, sem):
        pltpu.make_async_copy(col_b, seq_hbm.at[0, :, pl.ds(b0, BW)], sem).wait()

    def transpose_pool(rows_b, col_b):
        for f in range(D):
            cf = jnp.full((16,), f, jnp.int32)
            for g in range(BW // 16):
                v = plsc.load_gather(rows_b, [rowidx[g], cf])
                col_b[f, pl.ds(g * 16, 16)] = v
                plsc.addupdate(accv.at[f, pl.ds(g * 16, 16)], v)

    fire_gather(0, rows0, sg0)

    def pair_body(p, _):
        t0 = 2 * p
        @pl.when(p > 0)
        def _():
            drain_out(col0, sw0)
        drain_gather(rows0, sg0)
        fire_gather(t0 + 1, rows1, sg1)
        transpose_pool(rows0, col0)
        fire_out(t0, col0, sw0)
        @pl.when(p > 0)
        def _():
            drain_out(col1, sw1)
        drain_gather(rows1, sg1)

        @pl.when(p < L // 2 - 1)
        def _():
            fire_gather(t0 + 2, rows0, sg0)
        transpose_pool(rows1, col1)
        fire_out(t0 + 1, col1, sw1)
        return 0

    lax.fori_loop(0, L // 2, pair_body, 0)
    drain_out(col0, sw0)
    drain_out(col1, sw1)

    inv = 1.0 / L

    def scale_body(i, _):
        f = i // (BW // 16)
        o = (i % (BW // 16)) * 16
        accv[f, pl.ds(o, 16)] = accv[f, pl.ds(o, 16)] * inv
        return 0

    lax.fori_loop(0, D * BW // 16, scale_body, 0)
    pltpu.sync_copy(accv, pool_hbm.at[:, pl.ds(b0, BW)])


def kernel(input_ids, embeddings):
    ids_t = input_ids.T.astype(jnp.int32)
    seq_tfb, pooled_fb = _sc_embed(ids_t, embeddings)
    seq = jnp.transpose(seq_tfb, (2, 0, 1))
    pooled = pooled_fb.T
    return seq, pooled


# speedup vs baseline: 1.2927x; 1.2927x over previous
import functools

import jax
import jax.numpy as jnp
from jax import lax
from jax.experimental import pallas as pl
from jax.experimental.pallas import tpu as pltpu
from jax.experimental.pallas import tpu_sc as plsc

VOCAB = 1_000_000
D = 32
B = 4096
L = 200

NC = 2
NS = 16
NW = NC * NS
BW = B // NW

_mesh = plsc.VectorSubcoreMesh(core_axis_name="c", subcore_axis_name="s")


@functools.partial(
    pl.kernel,
    out_type=[
        jax.ShapeDtypeStruct((L, B, D), jnp.float32),
        jax.ShapeDtypeStruct((B, D), jnp.float32),
    ],
    mesh=_mesh,
    compiler_params=pltpu.CompilerParams(use_tc_tiling_on_sc=False),
    scratch_types=[
        pltpu.VMEM((L, BW), jnp.int32),
        pltpu.VMEM((BW, D), jnp.float32),
        pltpu.VMEM((BW, D), jnp.float32),
        pltpu.VMEM((BW, D), jnp.float32),
        pltpu.SemaphoreType.DMA,
        pltpu.SemaphoreType.DMA,
        pltpu.SemaphoreType.DMA,
        pltpu.SemaphoreType.DMA,
    ],
)
def _sc_embed(ids_hbm, table_hbm, seq_hbm, pool_hbm,
              idsv, rows0, rows1, accv, sg0, sg1, sw0, sw1):
    wid = lax.axis_index("s") * NC + lax.axis_index("c")
    b0 = wid * BW

    pltpu.sync_copy(ids_hbm.at[:, pl.ds(b0, BW)], idsv)

    NV = BW * D // 16

    zero = jnp.zeros((16,), jnp.float32)

    def zero_body(i, _):
        accv[i // 2, pl.ds((i % 2) * 16, 16)] = zero
        return 0

    lax.fori_loop(0, NV, zero_body, 0)

    def fire_gather(t, rows_b, sem):
        pltpu.async_copy(table_hbm.at[idsv.at[t]], rows_b, sem)

    def drain_gather(rows_b, sem):
        pltpu.make_async_copy(table_hbm.at[idsv.at[0]], rows_b, sem).wait()

    def fire_out(t, rows_b, sem):
        pltpu.async_copy(rows_b, seq_hbm.at[t, pl.ds(b0, BW)], sem)

    def drain_out(rows_b, sem):
        pltpu.make_async_copy(rows_b, seq_hbm.at[0, pl.ds(b0, BW)], sem).wait()

    def pool_add(rows_b):
        for b in range(BW):
            for h in range(2):
                s = pl.ds(h * 16, 16)
                accv[b, s] = accv[b, s] + rows_b[b, s]

    fire_gather(0, rows0, sg0)

    def pair_body(p, _):
        t0 = 2 * p
        @pl.when(p > 0)
        def _():
            drain_out(rows0, sw0)
        drain_gather(rows0, sg0)
        fire_gather(t0 + 1, rows1, sg1)
        pool_add(rows0)
        fire_out(t0, rows0, sw0)
        @pl.when(p > 0)
        def _():
            drain_out(rows1, sw1)
        drain_gather(rows1, sg1)

        @pl.when(p < L // 2 - 1)
        def _():
            fire_gather(t0 + 2, rows0, sg0)
        pool_add(rows1)
        fire_out(t0 + 1, rows1, sw1)
        return 0

    lax.fori_loop(0, L // 2, pair_body, 0)
    drain_out(rows0, sw0)
    drain_out(rows1, sw1)

    inv = 1.0 / L

    def scale_body(i, _):
        b = i // 2
        s = pl.ds((i % 2) * 16, 16)
        accv[b, s] = accv[b, s] * inv
        return 0

    lax.fori_loop(0, NV, scale_body, 0)
    pltpu.sync_copy(accv, pool_hbm.at[pl.ds(b0, BW)])


def kernel(input_ids, embeddings):
    ids_t = input_ids.T.astype(jnp.int32)
    seq_tbf, pooled = _sc_embed(ids_t, embeddings)
    seq = jnp.transpose(seq_tbf, (1, 0, 2))
    return seq, pooled
